# baseline (speedup 1.0000x reference)
import functools

import jax
import jax.numpy as jnp
from jax import lax
from jax.experimental import pallas as pl
from jax.experimental.pallas import tpu as pltpu
from jax.experimental.pallas import tpu_sc as plsc

NC = 2
NS = 16
L = 16
NW = NC * NS

NUM_IDX = 16384
EMBED = 64
PER_W = NUM_IDX // NW
CHUNK = 128
NCHUNK = PER_W // CHUNK
NCOL = EMBED // L


def _sc_partial_sums(idx3, table):
    mesh = plsc.VectorSubcoreMesh(core_axis_name="c", subcore_axis_name="s")

    @functools.partial(
        pl.kernel,
        out_type=jax.ShapeDtypeStruct((NW, EMBED), jnp.float32),
        mesh=mesh,
        scratch_types=[
            pltpu.VMEM((NCHUNK, CHUNK), jnp.int32),
            pltpu.VMEM((PER_W, EMBED), jnp.float32),
            pltpu.VMEM((EMBED,), jnp.float32),
            pltpu.SemaphoreType.DMA,
        ],
        compiler_params=pltpu.CompilerParams(use_tc_tiling_on_sc=False),
    )
    def k(idx_hbm, table_hbm, out_hbm, idx_v, rows_v, acc_v, sem):
        wid = lax.axis_index("s") * NC + lax.axis_index("c")
        pltpu.sync_copy(idx_hbm.at[wid], idx_v)
        copies = [
            pltpu.async_copy(
                table_hbm.at[idx_v.at[j]],
                rows_v.at[pl.ds(j * CHUNK, CHUNK)],
                sem,
            )
            for j in range(NCHUNK)
        ]
        for cp in copies:
            cp.wait()

        def body(r, carry):
            return tuple(
                carry[c] + rows_v[r, pl.ds(c * L, L)] for c in range(NCOL)
            )

        acc = lax.fori_loop(
            0, PER_W, body,
            tuple(jnp.zeros((L,), jnp.float32) for _ in range(NCOL)),
        )
        for c in range(NCOL):
            acc_v[pl.ds(c * L, L)] = acc[c]
        pltpu.sync_copy(acc_v, out_hbm.at[wid])

    return k(idx3, table)


def _tc_mlp(partials, W1, b1, W2, b2):
    def body(p_ref, w1_ref, b1_ref, w2_ref, b2_ref, o_ref):
        s = jnp.sum(p_ref[...], axis=0, keepdims=True)
        h = lax.dot_general(
            s, w1_ref[...], (((1,), (1,)), ((), ())),
            precision=lax.Precision.HIGHEST,
            preferred_element_type=jnp.float32,
        )
        h = jnp.maximum(h + b1_ref[...], 0.0)
        o_ref[0, 0] = jnp.sum(h * w2_ref[...]) + b2_ref[0, 0]

    return pl.pallas_call(
        body,
        out_shape=jax.ShapeDtypeStruct((1, 1), jnp.float32),
        in_specs=[
            pl.BlockSpec(memory_space=pltpu.VMEM),
            pl.BlockSpec(memory_space=pltpu.VMEM),
            pl.BlockSpec(memory_space=pltpu.VMEM),
            pl.BlockSpec(memory_space=pltpu.VMEM),
            pl.BlockSpec(memory_space=pltpu.SMEM),
        ],
        out_specs=pl.BlockSpec(memory_space=pltpu.SMEM),
    )(partials, W1, b1.reshape(1, -1), W2, b2.reshape(1, 1))


def kernel(indices, table, W1, b1, W2, b2):
    idx3 = indices.astype(jnp.int32).reshape(NW, NCHUNK, CHUNK)
    partials = _sc_partial_sums(idx3, table)
    out = _tc_mlp(partials, W1, b1, W2, b2)
    return out.reshape(1)


# speedup vs baseline: 1.7129x; 1.7129x over previous
import functools

import jax
import jax.numpy as jnp
from jax import lax
from jax.experimental import pallas as pl
from jax.experimental.pallas import tpu as pltpu
from jax.experimental.pallas import tpu_sc as plsc

NC = 2
NS = 16
L = 16
NW = NC * NS

NUM_IDX = 16384
EMBED = 64
PER_W = NUM_IDX // NW
BATCH = 64
NBATCH = PER_W // BATCH
NCOL = EMBED // L


def _sc_partial_sums(indices, table):
    mesh = plsc.VectorSubcoreMesh(core_axis_name="c", subcore_axis_name="s")

    @functools.partial(
        pl.kernel,
        out_type=jax.ShapeDtypeStruct((NW * EMBED,), jnp.float32),
        mesh=mesh,
        scratch_types=[
            pltpu.VMEM((PER_W,), jnp.int32),
            pltpu.VMEM((PER_W, EMBED), jnp.float32),
            pltpu.VMEM((EMBED,), jnp.float32),
            pltpu.SemaphoreType.DMA,
            pltpu.SemaphoreType.DMA,
        ],
    )
    def k(idx_hbm, table_hbm, out_hbm, idx_v, rows_v, acc_v, sem0, sem1):
        wid = lax.axis_index("s") * NC + lax.axis_index("c")
        base = wid * PER_W
        pltpu.sync_copy(idx_hbm.at[pl.ds(base, PER_W)], idx_v)
        sems = (sem0, sem1)

        def fire(b):
            sem = sems[b % 2]

            def one(g, _):
                r0 = b * BATCH + g * L
                vec = idx_v[pl.ds(r0, L)]
                for t in range(L):
                    pltpu.async_copy(
                        table_hbm.at[pl.ds(vec[t], 1)],
                        rows_v.at[pl.ds(r0 + t, 1)],
                        sem,
                    )
                return 0

            lax.fori_loop(0, BATCH // L, one, 0)

        def drain(b):
            pltpu.make_async_copy(
                table_hbm.at[pl.ds(0, BATCH)],
                rows_v.at[pl.ds(b * BATCH, BATCH)],
                sems[b % 2],
            ).wait()

        def reduce_batch(b, acc):
            def body(i, acc):
                r = b * BATCH + i
                return tuple(
                    acc[c] + rows_v[r, pl.ds(c * L, L)] for c in range(NCOL)
                )

            return lax.fori_loop(0, BATCH, body, acc)

        fire(0)
        acc = tuple(jnp.zeros((L,), jnp.float32) for _ in range(NCOL))
        for b in range(1, NBATCH):
            fire(b)
            drain(b - 1)
            acc = reduce_batch(b - 1, acc)
        drain(NBATCH - 1)
        acc = reduce_batch(NBATCH - 1, acc)

        for c in range(NCOL):
            acc_v[pl.ds(c * L, L)] = acc[c]
        pltpu.sync_copy(acc_v, out_hbm.at[pl.ds(wid * EMBED, EMBED)])

    return k(indices, table)


def _tc_mlp(partials, W1, b1, W2, b2):
    def body(p_ref, w1_ref, b1_ref, w2_ref, b2_ref, o_ref):
        s = jnp.sum(p_ref[...], axis=0, keepdims=True)
        h = lax.dot_general(
            s, w1_ref[...], (((1,), (1,)), ((), ())),
            precision=lax.Precision.HIGHEST,
            preferred_element_type=jnp.float32,
        )
        h = jnp.maximum(h + b1_ref[...], 0.0)
        o_ref[0, 0] = jnp.sum(h * w2_ref[...]) + b2_ref[0, 0]

    return pl.pallas_call(
        body,
        out_shape=jax.ShapeDtypeStruct((1, 1), jnp.float32),
        in_specs=[
            pl.BlockSpec(memory_space=pltpu.VMEM),
            pl.BlockSpec(memory_space=pltpu.VMEM),
            pl.BlockSpec(memory_space=pltpu.VMEM),
            pl.BlockSpec(memory_space=pltpu.VMEM),
            pl.BlockSpec(memory_space=pltpu.SMEM),
        ],
        out_specs=pl.BlockSpec(memory_space=pltpu.SMEM),
    )(partials, W1, b1.reshape(1, -1), W2, b2.reshape(1, 1))


def kernel(indices, table, W1, b1, W2, b2):
    partials = _sc_partial_sums(indices.astype(jnp.int32), table)
    out = _tc_mlp(partials.reshape(NW, EMBED), W1, b1, W2, b2)
    return out.reshape(1)
